# baseline (speedup 1.0000x reference)
import functools

import jax
import jax.numpy as jnp
from jax import lax
from jax.experimental import pallas as pl
from jax.experimental.pallas import tpu as pltpu
from jax.experimental.pallas import tpu_sc as plsc

N_NODES = 10000
N_EDGES = 320000
D_FEAT = 128
N_FILTERS = 128
N_RBF = 20

_LOG2 = 0.6931471805599453

_NC = 2
_NS = 16
_NW = _NC * _NS
_CHUNK = 64
_NCHUNKS = N_EDGES // _CHUNK
_N_PAD = 10240
_ROWS_PER_TILE = _N_PAD // _NS


def _ssp(v):
    return jax.nn.softplus(v) - _LOG2


def _h_body(x_ref, w_ref, o_ref):
    o_ref[...] = jnp.dot(x_ref[...], w_ref[...],
                         preferred_element_type=jnp.float32)


def _input_to_feature(x, w_in):
    rb = 1000
    return pl.pallas_call(
        _h_body,
        grid=(N_NODES // rb,),
        in_specs=[
            pl.BlockSpec((rb, D_FEAT), lambda i: (i, 0)),
            pl.BlockSpec((D_FEAT, N_FILTERS), lambda i: (0, 0)),
        ],
        out_specs=pl.BlockSpec((rb, N_FILTERS), lambda i: (i, 0)),
        out_shape=jax.ShapeDtypeStruct((N_NODES, N_FILTERS), jnp.float32),
    )(x, w_in)


def _filter_body(f_ref, c_ref, w1_ref, b1_ref, w2_ref, b2_ref, o_ref):
    t = jnp.dot(f_ref[:, 0, :], w1_ref[...], preferred_element_type=jnp.float32)
    t = _ssp(t + b1_ref[...])
    t = jnp.dot(t, w2_ref[...], preferred_element_type=jnp.float32)
    o_ref[...] = (t + b2_ref[...]) * c_ref[...]


def _filter_network(f3d, cutoff, wf1, bf1, wf2, bf2):
    be = 4000
    return pl.pallas_call(
        _filter_body,
        grid=(N_EDGES // be,),
        in_specs=[
            pl.BlockSpec((be, 1, N_RBF), lambda i: (i, 0, 0)),
            pl.BlockSpec((be, 1), lambda i: (i, 0)),
            pl.BlockSpec((N_RBF, N_FILTERS), lambda i: (0, 0)),
            pl.BlockSpec((1, N_FILTERS), lambda i: (0, 0)),
            pl.BlockSpec((N_FILTERS, N_FILTERS), lambda i: (0, 0)),
            pl.BlockSpec((1, N_FILTERS), lambda i: (0, 0)),
        ],
        out_specs=pl.BlockSpec((be, N_FILTERS), lambda i: (i, 0)),
        out_shape=jax.ShapeDtypeStruct((N_EDGES, N_FILTERS), jnp.float32),
    )(f3d, cutoff, wf1, bf1, wf2, bf2)


_NB = (_NCHUNKS // _NW) & ~1
_NTAIL = _NCHUNKS - _NB * _NW
assert _NTAIL <= _NW


def _sc_body(h_hbm, wij_hbm, pair_hbm, zeros_hbm, out0_hbm, out1_hbm,
             idxj_v0, idxi_v0, rows_v0, wij_v0,
             idxj_v1, idxi_v1, rows_v1, wij_v1,
             acc_sh,
             semj0, semi0, semw0, semg0, sems0,
             semj1, semi1, semw1, semg1, sems1):
    c = lax.axis_index("c")
    s = lax.axis_index("s")
    wid = c * _NS + s

    slot0 = (idxj_v0, idxi_v0, rows_v0, wij_v0, semj0, semi0, semw0, semg0, sems0)
    slot1 = (idxj_v1, idxi_v1, rows_v1, wij_v1, semj1, semi1, semw1, semg1, sems1)

    pltpu.sync_copy(zeros_hbm, acc_sh.at[pl.ds(s * _ROWS_PER_TILE, _ROWS_PER_TILE)])
    plsc.subcore_barrier()

    def start_fetch(t, slot):
        idxj_v, idxi_v, rows_v, wij_v, semj, semi, semw, semg, sems = slot
        base = (wid + t * _NW) * _CHUNK
        pltpu.async_copy(pair_hbm.at[pl.ds(N_EDGES + base, _CHUNK)], idxj_v, semj)
        pltpu.async_copy(pair_hbm.at[pl.ds(base, _CHUNK)], idxi_v, semi)
        pltpu.async_copy(wij_hbm.at[pl.ds(base, _CHUNK)], wij_v, semw)

    def wait_scatter(slot):
        idxj_v, idxi_v, rows_v, wij_v, semj, semi, semw, semg, sems = slot
        pltpu.make_async_copy(rows_v, acc_sh.at[idxi_v], sems).wait()

    def wait_fetch_j(slot):
        idxj_v, idxi_v, rows_v, wij_v, semj, semi, semw, semg, sems = slot
        pltpu.make_async_copy(pair_hbm.at[pl.ds(0, _CHUNK)], idxj_v, semj).wait()

    def gather(slot):
        idxj_v, idxi_v, rows_v, wij_v, semj, semi, semw, semg, sems = slot
        pltpu.async_copy(h_hbm.at[idxj_v], rows_v, semg)

    def wait_gather(slot):
        idxj_v, idxi_v, rows_v, wij_v, semj, semi, semw, semg, sems = slot
        pltpu.make_async_copy(h_hbm.at[idxj_v], rows_v, semg).wait()

    def mul_scatter(slot):
        idxj_v, idxi_v, rows_v, wij_v, semj, semi, semw, semg, sems = slot
        pltpu.make_async_copy(wij_hbm.at[pl.ds(0, _CHUNK)], wij_v, semw).wait()

        @plsc.parallel_loop(0, _CHUNK, unroll=4)
        def _rows(r):
            for q in range(D_FEAT // 16):
                sl = pl.ds(q * 16, 16)
                rows_v[r, sl] = rows_v[r, sl] * wij_v[r, sl]

        pltpu.make_async_copy(pair_hbm.at[pl.ds(0, _CHUNK)], idxi_v, semi).wait()
        pltpu.async_copy(rows_v, acc_sh.at[idxi_v], sems, add=True)

    start_fetch(0, slot0)
    wait_fetch_j(slot0)
    gather(slot0)

    @pl.loop(0, _NB // 2)
    def _pairs(p):
        @pl.when(p > 0)
        def _():
            wait_scatter(slot1)

        start_fetch(2 * p + 1, slot1)
        wait_gather(slot0)
        wait_fetch_j(slot1)
        gather(slot1)
        mul_scatter(slot0)
        wait_gather(slot1)
        wait_scatter(slot0)

        @pl.when(p + 1 < _NB // 2)
        def _():
            start_fetch(2 * p + 2, slot0)
            wait_fetch_j(slot0)
            gather(slot0)

        mul_scatter(slot1)

    wait_scatter(slot1)

    @pl.when(wid < _NTAIL)
    def _tail():
        base = (_NB * _NW + wid) * _CHUNK
        pltpu.sync_copy(pair_hbm.at[pl.ds(N_EDGES + base, _CHUNK)], idxj_v0)
        pltpu.sync_copy(pair_hbm.at[pl.ds(base, _CHUNK)], idxi_v0)
        pltpu.sync_copy(wij_hbm.at[pl.ds(base, _CHUNK)], wij_v0)
        pltpu.async_copy(h_hbm.at[idxj_v0], rows_v0, semg0).wait()

        @pl.loop(0, _CHUNK)
        def _rows(r):
            for q in range(D_FEAT // 16):
                sl = pl.ds(q * 16, 16)
                rows_v0[r, sl] = rows_v0[r, sl] * wij_v0[r, sl]

        pltpu.sync_copy(rows_v0, acc_sh.at[idxi_v0], add=True)

    plsc.subcore_barrier()
    row0 = pl.ds(s * _ROWS_PER_TILE, _ROWS_PER_TILE)

    @pl.when(c == 0)
    def _dump0():
        pltpu.sync_copy(acc_sh.at[row0], out0_hbm.at[row0])

    @pl.when(c == 1)
    def _dump1():
        pltpu.sync_copy(acc_sh.at[row0], out1_hbm.at[row0])


def _sc_scatter(h, wij, pairlist, zeros_tile):
    mesh = plsc.VectorSubcoreMesh(core_axis_name="c", subcore_axis_name="s")
    k = pl.kernel(
        _sc_body,
        out_type=(jax.ShapeDtypeStruct((_N_PAD, D_FEAT), jnp.float32),
                  jax.ShapeDtypeStruct((_N_PAD, D_FEAT), jnp.float32)),
        mesh=mesh,
        compiler_params=pltpu.CompilerParams(use_tc_tiling_on_sc=True),
        scratch_types=(
            [pltpu.VMEM((_CHUNK,), jnp.int32),
             pltpu.VMEM((_CHUNK,), jnp.int32),
             pltpu.VMEM((_CHUNK, D_FEAT), jnp.float32),
             pltpu.VMEM((_CHUNK, D_FEAT), jnp.float32)] * 2
            + [pltpu.VMEM_SHARED((_N_PAD, D_FEAT), jnp.float32)]
            + [pltpu.SemaphoreType.DMA] * 10
        ),
    )
    return k(h, wij, pairlist.reshape(-1), zeros_tile)


def _out_body(p0_ref, p1_ref, w1_ref, b1_ref, w2_ref, b2_ref, o_ref):
    t = p0_ref[...] + p1_ref[...]
    t = _ssp(jnp.dot(t, w1_ref[...], preferred_element_type=jnp.float32)
             + b1_ref[...])
    o_ref[...] = jnp.dot(t, w2_ref[...], preferred_element_type=jnp.float32) \
        + b2_ref[...]


def _output_network(p0, p1, wo1, bo1, wo2, bo2):
    rb = 1000
    return pl.pallas_call(
        _out_body,
        grid=(N_NODES // rb,),
        in_specs=[
            pl.BlockSpec((rb, N_FILTERS), lambda i: (i, 0)),
            pl.BlockSpec((rb, N_FILTERS), lambda i: (i, 0)),
            pl.BlockSpec((N_FILTERS, D_FEAT), lambda i: (0, 0)),
            pl.BlockSpec((1, D_FEAT), lambda i: (0, 0)),
            pl.BlockSpec((D_FEAT, D_FEAT), lambda i: (0, 0)),
            pl.BlockSpec((1, D_FEAT), lambda i: (0, 0)),
        ],
        out_specs=pl.BlockSpec((rb, D_FEAT), lambda i: (i, 0)),
        out_shape=jax.ShapeDtypeStruct((N_NODES, D_FEAT), jnp.float32),
    )(p0, p1, wo1, bo1, wo2, bo2)


def kernel(x, pairlist, f_ij, f_ij_cutoff, W_in, Wf1, bf1, Wf2, bf2,
           Wo1, bo1, Wo2, bo2):
    h = _input_to_feature(x, W_in)
    wij = _filter_network(f_ij, f_ij_cutoff, Wf1, bf1.reshape(1, -1),
                          Wf2, bf2.reshape(1, -1))
    zeros_tile = jnp.zeros((_ROWS_PER_TILE, D_FEAT), jnp.float32)
    p0, p1 = _sc_scatter(h, wij, pairlist, zeros_tile)
    return _output_network(p0, p1, Wo1, bo1.reshape(1, -1),
                           Wo2, bo2.reshape(1, -1))


# speedup vs baseline: 1.5541x; 1.5541x over previous
import functools

import jax
import jax.numpy as jnp
from jax import lax
from jax.experimental import pallas as pl
from jax.experimental.pallas import tpu as pltpu
from jax.experimental.pallas import tpu_sc as plsc

N_NODES = 10000
N_EDGES = 320000
D_FEAT = 128
N_FILTERS = 128
N_RBF = 20

_LOG2 = 0.6931471805599453

_NC = 2
_NS = 16
_NW = _NC * _NS
_CHUNK = 64
_NCHUNKS = N_EDGES // _CHUNK
_N_PAD = 10240
_ROWS_PER_TILE = _N_PAD // _NS


def _ssp(v):
    return jax.nn.softplus(v) - _LOG2


def _h_body(x_ref, w_ref, o_ref):
    o_ref[...] = jnp.dot(x_ref[...], w_ref[...],
                         preferred_element_type=jnp.float32)


def _input_to_feature(x, w_in):
    rb = 1000
    return pl.pallas_call(
        _h_body,
        grid=(N_NODES // rb,),
        in_specs=[
            pl.BlockSpec((rb, D_FEAT), lambda i: (i, 0)),
            pl.BlockSpec((D_FEAT, N_FILTERS), lambda i: (0, 0)),
        ],
        out_specs=pl.BlockSpec((rb, N_FILTERS), lambda i: (i, 0)),
        out_shape=jax.ShapeDtypeStruct((N_NODES, N_FILTERS), jnp.float32),
    )(x, w_in)


def _filter_body(f_ref, c_ref, w1_ref, b1_ref, w2_ref, b2_ref, o_ref):
    t = jnp.dot(f_ref[...], w1_ref[...], preferred_element_type=jnp.float32)
    t = _ssp(t + b1_ref[...])
    t = jnp.dot(t, w2_ref[...], preferred_element_type=jnp.float32)
    o_ref[...] = (t + b2_ref[...]) * c_ref[...]


def _filter_network(f2d, cutoff, wf1, bf1, wf2, bf2):
    be = 4000
    return pl.pallas_call(
        _filter_body,
        grid=(N_EDGES // be,),
        in_specs=[
            pl.BlockSpec((be, N_RBF), lambda i: (i, 0)),
            pl.BlockSpec((be, 1), lambda i: (i, 0)),
            pl.BlockSpec((N_RBF, N_FILTERS), lambda i: (0, 0)),
            pl.BlockSpec((1, N_FILTERS), lambda i: (0, 0)),
            pl.BlockSpec((N_FILTERS, N_FILTERS), lambda i: (0, 0)),
            pl.BlockSpec((1, N_FILTERS), lambda i: (0, 0)),
        ],
        out_specs=pl.BlockSpec((be, N_FILTERS), lambda i: (i, 0)),
        out_shape=jax.ShapeDtypeStruct((N_EDGES, N_FILTERS), jnp.float32),
    )(f2d, cutoff, wf1, bf1, wf2, bf2)


_NB = (_NCHUNKS // _NW) & ~1
_NTAIL = _NCHUNKS - _NB * _NW
assert _NTAIL <= _NW


def _sc_body(h_hbm, wij_hbm, pair_hbm, zeros_hbm, out0_hbm, out1_hbm,
             idxj_v0, idxi_v0, rows_v0, wij_v0,
             idxj_v1, idxi_v1, rows_v1, wij_v1,
             acc_sh,
             semj0, semi0, semw0, semg0, sems0,
             semj1, semi1, semw1, semg1, sems1):
    c = lax.axis_index("c")
    s = lax.axis_index("s")
    wid = c * _NS + s

    slot0 = (idxj_v0, idxi_v0, rows_v0, wij_v0, semj0, semi0, semw0, semg0, sems0)
    slot1 = (idxj_v1, idxi_v1, rows_v1, wij_v1, semj1, semi1, semw1, semg1, sems1)

    pltpu.sync_copy(zeros_hbm, acc_sh.at[pl.ds(s * _ROWS_PER_TILE, _ROWS_PER_TILE)])
    plsc.subcore_barrier()

    def start_fetch(t, slot):
        idxj_v, idxi_v, rows_v, wij_v, semj, semi, semw, semg, sems = slot
        base = (wid + t * _NW) * _CHUNK
        pltpu.async_copy(pair_hbm.at[pl.ds(N_EDGES + base, _CHUNK)], idxj_v, semj)
        pltpu.async_copy(pair_hbm.at[pl.ds(base, _CHUNK)], idxi_v, semi)
        pltpu.async_copy(wij_hbm.at[pl.ds(base, _CHUNK)], wij_v, semw)

    def wait_scatter(slot):
        idxj_v, idxi_v, rows_v, wij_v, semj, semi, semw, semg, sems = slot
        pltpu.make_async_copy(rows_v, acc_sh.at[idxi_v], sems).wait()

    def wait_fetch_j(slot):
        idxj_v, idxi_v, rows_v, wij_v, semj, semi, semw, semg, sems = slot
        pltpu.make_async_copy(pair_hbm.at[pl.ds(0, _CHUNK)], idxj_v, semj).wait()

    def gather(slot):
        idxj_v, idxi_v, rows_v, wij_v, semj, semi, semw, semg, sems = slot
        pltpu.async_copy(h_hbm.at[idxj_v], rows_v, semg)

    def wait_gather(slot):
        idxj_v, idxi_v, rows_v, wij_v, semj, semi, semw, semg, sems = slot
        pltpu.make_async_copy(h_hbm.at[idxj_v], rows_v, semg).wait()

    def mul_scatter(slot):
        idxj_v, idxi_v, rows_v, wij_v, semj, semi, semw, semg, sems = slot
        pltpu.make_async_copy(wij_hbm.at[pl.ds(0, _CHUNK)], wij_v, semw).wait()

        @plsc.parallel_loop(0, _CHUNK, unroll=4)
        def _rows(r):
            for q in range(D_FEAT // 16):
                sl = pl.ds(q * 16, 16)
                rows_v[r, sl] = rows_v[r, sl] * wij_v[r, sl]

        pltpu.make_async_copy(pair_hbm.at[pl.ds(0, _CHUNK)], idxi_v, semi).wait()
        pltpu.async_copy(rows_v, acc_sh.at[idxi_v], sems, add=True)

    start_fetch(0, slot0)
    wait_fetch_j(slot0)
    gather(slot0)

    @pl.loop(0, _NB // 2)
    def _pairs(p):
        @pl.when(p > 0)
        def _():
            wait_scatter(slot1)

        start_fetch(2 * p + 1, slot1)
        wait_gather(slot0)
        wait_fetch_j(slot1)
        gather(slot1)
        mul_scatter(slot0)
        wait_gather(slot1)
        wait_scatter(slot0)

        @pl.when(p + 1 < _NB // 2)
        def _():
            start_fetch(2 * p + 2, slot0)
            wait_fetch_j(slot0)
            gather(slot0)

        mul_scatter(slot1)

    wait_scatter(slot1)

    @pl.when(wid < _NTAIL)
    def _tail():
        base = (_NB * _NW + wid) * _CHUNK
        pltpu.sync_copy(pair_hbm.at[pl.ds(N_EDGES + base, _CHUNK)], idxj_v0)
        pltpu.sync_copy(pair_hbm.at[pl.ds(base, _CHUNK)], idxi_v0)
        pltpu.sync_copy(wij_hbm.at[pl.ds(base, _CHUNK)], wij_v0)
        pltpu.async_copy(h_hbm.at[idxj_v0], rows_v0, semg0).wait()

        @pl.loop(0, _CHUNK)
        def _rows(r):
            for q in range(D_FEAT // 16):
                sl = pl.ds(q * 16, 16)
                rows_v0[r, sl] = rows_v0[r, sl] * wij_v0[r, sl]

        pltpu.sync_copy(rows_v0, acc_sh.at[idxi_v0], add=True)

    plsc.subcore_barrier()
    row0 = pl.ds(s * _ROWS_PER_TILE, _ROWS_PER_TILE)

    @pl.when(c == 0)
    def _dump0():
        pltpu.sync_copy(acc_sh.at[row0], out0_hbm.at[row0])

    @pl.when(c == 1)
    def _dump1():
        pltpu.sync_copy(acc_sh.at[row0], out1_hbm.at[row0])


def _sc_scatter(h, wij, pairlist, zeros_tile):
    mesh = plsc.VectorSubcoreMesh(core_axis_name="c", subcore_axis_name="s")
    k = pl.kernel(
        _sc_body,
        out_type=(jax.ShapeDtypeStruct((_N_PAD, D_FEAT), jnp.float32),
                  jax.ShapeDtypeStruct((_N_PAD, D_FEAT), jnp.float32)),
        mesh=mesh,
        compiler_params=pltpu.CompilerParams(use_tc_tiling_on_sc=True),
        scratch_types=(
            [pltpu.VMEM((_CHUNK,), jnp.int32),
             pltpu.VMEM((_CHUNK,), jnp.int32),
             pltpu.VMEM((_CHUNK, D_FEAT), jnp.float32),
             pltpu.VMEM((_CHUNK, D_FEAT), jnp.float32)] * 2
            + [pltpu.VMEM_SHARED((_N_PAD, D_FEAT), jnp.float32)]
            + [pltpu.SemaphoreType.DMA] * 10
        ),
    )
    return k(h, wij, pairlist.reshape(-1), zeros_tile)


def _out_body(p0_ref, p1_ref, w1_ref, b1_ref, w2_ref, b2_ref, o_ref):
    t = p0_ref[...] + p1_ref[...]
    t = _ssp(jnp.dot(t, w1_ref[...], preferred_element_type=jnp.float32)
             + b1_ref[...])
    o_ref[...] = jnp.dot(t, w2_ref[...], preferred_element_type=jnp.float32) \
        + b2_ref[...]


def _output_network(p0, p1, wo1, bo1, wo2, bo2):
    rb = 1000
    return pl.pallas_call(
        _out_body,
        grid=(N_NODES // rb,),
        in_specs=[
            pl.BlockSpec((rb, N_FILTERS), lambda i: (i, 0)),
            pl.BlockSpec((rb, N_FILTERS), lambda i: (i, 0)),
            pl.BlockSpec((N_FILTERS, D_FEAT), lambda i: (0, 0)),
            pl.BlockSpec((1, D_FEAT), lambda i: (0, 0)),
            pl.BlockSpec((D_FEAT, D_FEAT), lambda i: (0, 0)),
            pl.BlockSpec((1, D_FEAT), lambda i: (0, 0)),
        ],
        out_specs=pl.BlockSpec((rb, D_FEAT), lambda i: (i, 0)),
        out_shape=jax.ShapeDtypeStruct((N_NODES, D_FEAT), jnp.float32),
    )(p0, p1, wo1, bo1, wo2, bo2)


def kernel(x, pairlist, f_ij, f_ij_cutoff, W_in, Wf1, bf1, Wf2, bf2,
           Wo1, bo1, Wo2, bo2):
    h = _input_to_feature(x, W_in)
    wij = _filter_network(f_ij.reshape(N_EDGES, N_RBF),
                          f_ij_cutoff, Wf1,
                          bf1.reshape(1, -1), Wf2, bf2.reshape(1, -1))
    zeros_tile = jnp.zeros((_ROWS_PER_TILE, D_FEAT), jnp.float32)
    p0, p1 = _sc_scatter(h, wij, pairlist, zeros_tile)
    return _output_network(p0, p1, Wo1, bo1.reshape(1, -1),
                           Wo2, bo2.reshape(1, -1))
